# Initial kernel scaffold; baseline (speedup 1.0000x reference)
import jax
import jax.numpy as jnp
from jax.experimental import pallas as pl


def kernel(input_ids, attention_mask, embed_table):
    raise NotImplementedError("write your pallas kernel here")


# speedup vs baseline: 1.4288x; 1.4288x over previous
import functools

import jax
import jax.numpy as jnp
from jax import lax
from jax.experimental import pallas as pl
from jax.experimental.pallas import tpu as pltpu
from jax.experimental.pallas import tpu_sc as plsc

_B = 64
_L = 512
_D = 3584

_NC = 2
_NS = 16
_NW = _NC * _NS
_BPW = _B // _NW
_LANES = 16
_ND = _D // _LANES
_K = 32
_NCHUNK = _L // _K
_U = 8

_mesh = plsc.VectorSubcoreMesh(core_axis_name="c", subcore_axis_name="s")


@functools.partial(
    pl.kernel,
    out_type=jax.ShapeDtypeStruct((_B, _D), jnp.float32),
    mesh=_mesh,
    scratch_types=[
        pltpu.VMEM((_L,), jnp.int32),
        pltpu.VMEM((_L,), jnp.int32),
        pltpu.VMEM((_L,), jnp.float32),
        pltpu.VMEM((_K, _D), jnp.float32),
        pltpu.VMEM((_D,), jnp.float32),
        pltpu.SemaphoreType.DMA,
    ],
)
def _pooled_embedding(ids_hbm, mask_hbm, table_hbm, out_hbm,
                      idx_v, mi_v, mf_v, buf_v, acc_v, sem):
    wid = lax.axis_index("s") * _NC + lax.axis_index("c")

    for bi in range(_BPW):
        b = wid * _BPW + bi
        pltpu.sync_copy(ids_hbm.at[b], idx_v)
        pltpu.sync_copy(mask_hbm.at[b], mi_v)

        def mask_cvt(i, _):
            sl = pl.ds(i * _LANES, _LANES)
            mf_v[sl] = mi_v[sl].astype(jnp.float32)
            return 0
        lax.fori_loop(0, _L // _LANES, mask_cvt, 0)

        def zero_acc(i, _):
            acc_v[pl.ds(i * _LANES, _LANES)] = jnp.zeros((_LANES,), jnp.float32)
            return 0
        lax.fori_loop(0, _ND, zero_acc, 0)

        def chunk_body(s, _):
            base = pl.multiple_of(s * _K, 8)
            pltpu.async_copy(
                table_hbm.at[idx_v.at[pl.ds(base, _K)]], buf_v, sem
            ).wait()

            def dgroup_body(dg, _):
                d0 = dg * _U * _LANES
                accs = tuple(
                    acc_v[pl.ds(d0 + u * _LANES, _LANES)] for u in range(_U)
                )

                def row16_body(g, accs):
                    mvec = mf_v[pl.ds(base + g * _LANES, _LANES)]
                    for j in range(_LANES):
                        r = g * _LANES + j
                        m = mvec[j]
                        accs = tuple(
                            accs[u]
                            + buf_v[r, pl.ds(d0 + u * _LANES, _LANES)] * m
                            for u in range(_U)
                        )
                    return accs

                accs = lax.fori_loop(0, _K // _LANES, row16_body, accs)
                for u in range(_U):
                    acc_v[pl.ds(d0 + u * _LANES, _LANES)] = accs[u]
                return 0

            lax.fori_loop(0, _ND // _U, dgroup_body, 0)
            return 0

        lax.fori_loop(0, _NCHUNK, chunk_body, 0)

        def mask_sum(i, acc):
            return acc + mf_v[pl.ds(i * _LANES, _LANES)]
        msum = lax.fori_loop(0, _L // _LANES, mask_sum,
                             jnp.zeros((_LANES,), jnp.float32))
        mtot = msum[0]
        for j in range(1, _LANES):
            mtot = mtot + msum[j]
        total = jnp.maximum(mtot, 1e-9)
        tvec = jnp.full((_LANES,), 1.0, jnp.float32) * total
        inv = jnp.full((_LANES,), 1.0, jnp.float32) / tvec

        def scale_body(i, _):
            sl = pl.ds(i * _LANES, _LANES)
            acc_v[sl] = acc_v[sl] * inv
            return 0
        lax.fori_loop(0, _ND, scale_body, 0)

        pltpu.sync_copy(acc_v, out_hbm.at[b])


def kernel(input_ids, attention_mask, embed_table):
    return _pooled_embedding(input_ids, attention_mask, embed_table)
